# baseline (speedup 1.0000x reference)
import functools

import numpy as np
import jax
import jax.numpy as jnp
from jax import lax
from jax.experimental import pallas as pl
from jax.experimental.pallas import tpu as pltpu
from jax.experimental.pallas import tpu_sc as plsc

N = 10000
E = 160000
MUL = 8
D_IN = 32
D_MSG = 48
ALPHA = float(1.0 / np.sqrt(128.0))
INV_SQRT3 = float(1.0 / np.sqrt(3.0))
EPS = 1e-5

NCORES = 2
NSUB = 16
NW = NCORES * NSUB
STRIP = 125
STRIPS = E // STRIP
STRIPS_PER_W = STRIPS // NW
GROUP = 8
GROUPS = STRIPS_PER_W // GROUP
ROWS_PER_GROUP = STRIP * GROUP
ROWS_PER_W = STRIP * STRIPS_PER_W
NODE_ROWS_PER_S = N // NSUB

PK = 4
RPB = 800
GRID = (E // PK) // RPB

HI = jax.lax.Precision.HIGHEST


def _build_consts():
    perm = list(range(8)) + [8 + 3 * i + m for m in range(3) for i in range(8)]
    eye8 = np.eye(8, dtype=np.float32)
    cay = np.zeros((4, 96), np.float32)
    for i in range(8):
        cay[0, i] = 1.0
    for m in range(3):
        for i in range(8):
            cay[1 + m, 8 + m * 8 + i] = INV_SQRT3
        for j in range(8):
            cay[1 + m, 32 + m * 8 + j] = ALPHA
    for k in range(24):
        cay[0, 64 + k] = 1.0
    cm1coef = np.zeros((4, 32), np.float32)
    for i in range(8):
        cm1coef[0, i] = 1.0
    for m in range(3):
        for i in range(8):
            cm1coef[1 + m, 8 + m * 8 + i] = INV_SQRT3
    cy = np.zeros((4, 56), np.float32)
    for m in range(3):
        for j in range(8):
            cy[1 + m, 8 + m * 8 + j] = ALPHA
    for k in range(24):
        cy[0, 32 + k] = ALPHA
    cm1x = np.zeros((32, 384), np.float32)
    cm2x = np.zeros((32, 384), np.float32)
    for i in range(8):
        for j in range(8):
            cm1x[i, i * 8 + j] = 1.0
            cm2x[i, 128 + i * 8 + j] = 1.0
            for m in range(3):
                cm1x[8 + m * 8 + i, 64 + i * 8 + j] = 1.0
                cm2x[8 + m * 8 + i, 192 + m * 64 + i * 8 + j] = 1.0
    cq56 = np.zeros((384, 56), np.float32)
    for i in range(8):
        for j in range(8):
            cq56[0 * 64 + i * 8 + j, j] = ALPHA
            cq56[1 * 64 + i * 8 + j, j] = ALPHA
            for m in range(3):
                cq56[128 + i * 8 + j, 8 + m * 8 + j] = 1.0
                cq56[192 + m * 64 + i * 8 + j, 32 + m * 8 + j] = 1.0
    cos_ = np.zeros((56, 48), np.float32)
    cfv = np.zeros((56, 48), np.float32)
    for j in range(8):
        cos_[j, j] = 1.0
        for m in range(3):
            cfv[8 + m * 8 + j, 8 + m * 8 + j] = 1.0
            cfv[32 + m * 8 + j, 8 + m * 8 + j] = 1.0
    onesb = np.zeros((1, PK * D_MSG), np.float32)
    for k in range(PK):
        onesb[0, k * D_MSG + 32:(k + 1) * D_MSG] = 1.0
    gm = np.kron(np.full((3, 3), 1.0 / 3.0, np.float32), eye8)
    pout = np.zeros((24, 24), np.float32)
    for i in range(8):
        for m in range(3):
            pout[m * 8 + i, 3 * i + m] = 1.0

    def bd4(mat):
        return np.kron(np.eye(PK, dtype=np.float32), mat)

    cmx = np.vstack([bd4(cm1x), bd4(cm2x)])
    return (np.array(perm, np.int32), bd4(cm1coef), bd4(cy), cmx,
            bd4(cq56), bd4(cos_), bd4(cfv), onesb, gm, pout)


(_PERM, _CSHM, _CY, _CMX, _CQ56, _COS, _CFV, _ONESB,
 _GM, _POUT) = _build_consts()


def _gather_body(h_hbm, idx_hbm, x_hbm, idx_v, rows_v, sem):
    c = lax.axis_index("c")
    s = lax.axis_index("s")
    wid = s * NCORES + c
    pltpu.sync_copy(idx_hbm.at[pl.ds(wid * STRIPS_PER_W, STRIPS_PER_W)], idx_v)

    def group(g, carry):
        cps = [pltpu.async_copy(h_hbm.at[idx_v.at[g * GROUP + j]],
                                rows_v.at[pl.ds(j * STRIP, STRIP)], sem)
               for j in range(GROUP)]
        for cp in cps:
            cp.wait()
        row0 = wid * ROWS_PER_W + g * ROWS_PER_GROUP
        pltpu.sync_copy(rows_v, x_hbm.at[pl.ds(row0, ROWS_PER_GROUP)])
        return carry

    lax.fori_loop(0, GROUPS, group, 0)


@jax.jit
def _sc_gather(h_p, src2d):
    fn = pl.kernel(
        _gather_body,
        out_type=jax.ShapeDtypeStruct((E, D_IN), jnp.float32),
        mesh=plsc.VectorSubcoreMesh(core_axis_name="c", subcore_axis_name="s",
                                    num_cores=NCORES, num_subcores=NSUB),
        scratch_types=[
            pltpu.VMEM((STRIPS_PER_W, STRIP), jnp.int32),
            pltpu.VMEM((ROWS_PER_GROUP, D_IN), jnp.float32),
            pltpu.SemaphoreType.DMA,
        ],
        compiler_params=pltpu.CompilerParams(use_tc_tiling_on_sc=False),
    )
    return fn(h_p, src2d)


def _scatter_body(msg_hbm, dst_hbm, zeros_hbm, out_hbm, idx_v, msg_v, stage_v, acc_sh):
    c = lax.axis_index("c")
    s = lax.axis_index("s")
    wid = s * NCORES + c

    @pl.when(s == 0)
    def _():
        pltpu.sync_copy(zeros_hbm, acc_sh)

    plsc.subcore_barrier()
    pltpu.sync_copy(dst_hbm.at[pl.ds(wid * STRIPS_PER_W, STRIPS_PER_W)], idx_v)

    def group(g, carry):
        row0 = wid * ROWS_PER_W + g * ROWS_PER_GROUP
        pltpu.sync_copy(msg_hbm.at[pl.ds(row0, ROWS_PER_GROUP)], msg_v)
        for j in range(GROUP):
            pltpu.sync_copy(msg_v.at[pl.ds(j * STRIP, STRIP)],
                            acc_sh.at[idx_v.at[g * GROUP + j]], add=True)
        return carry

    lax.fori_loop(0, GROUPS, group, 0)
    plsc.subcore_barrier()
    pltpu.sync_copy(acc_sh.at[pl.ds(s * NODE_ROWS_PER_S, NODE_ROWS_PER_S)], stage_v)
    pltpu.sync_copy(stage_v, out_hbm.at[c, pl.ds(s * NODE_ROWS_PER_S, NODE_ROWS_PER_S)])


@jax.jit
def _sc_scatter(msg, dst2d, zeros_nd):
    fn = pl.kernel(
        _scatter_body,
        out_type=jax.ShapeDtypeStruct((NCORES, N, D_MSG), jnp.float32),
        mesh=plsc.VectorSubcoreMesh(core_axis_name="c", subcore_axis_name="s",
                                    num_cores=NCORES, num_subcores=NSUB),
        scratch_types=[
            pltpu.VMEM((STRIPS_PER_W, STRIP), jnp.int32),
            pltpu.VMEM((ROWS_PER_GROUP, D_MSG), jnp.float32),
            pltpu.VMEM((NODE_ROWS_PER_S, D_MSG), jnp.float32),
            pltpu.VMEM_SHARED((N, D_MSG), jnp.float32),
        ],
        compiler_params=pltpu.CompilerParams(use_tc_tiling_on_sc=False),
    )
    return fn(msg, dst2d, zeros_nd)


def _edge_body(ef_ref, x_ref, sh_ref, w1, b1, w2, b2, w3, b3,
               cshm, cy, cmx, cq56, cos_, cfv, onesb, out_ref):
    ef = ef_ref[...]
    xx = x_ref[...]
    shv = sh_ref[...]
    a = ef @ w1[...] + b1[...]
    a = a * jax.nn.sigmoid(a)
    a = a @ w2[...] + b2[...]
    a = a * jax.nn.sigmoid(a)
    w = a @ w3[...] + b3[...]

    shm = shv @ cshm[...]
    ymul = shv @ cy[...]
    m1 = xx * shm
    us = jnp.concatenate([m1, xx], axis=1) @ cmx[...]
    red = (w * us) @ cq56[...]
    out = red @ cos_[...] + (red * ymul) @ cfv[...] + onesb[...]
    out_ref[...] = out


@jax.jit
def _tc_edges(ef, x, sh, W1, b1, W2, b2, W3, b3):
    full = lambda shape: pl.BlockSpec(shape, lambda i: (0, 0))
    eye = jnp.eye(PK, dtype=jnp.float32)
    w1b = jnp.kron(eye, W1)
    w2b = jnp.kron(eye, W2)
    w3p = jnp.concatenate([W3[:, 0:192], jnp.tile(W3[:, 192:256], (1, 3))], axis=1)
    w3b = jnp.kron(eye, w3p)
    b3p = jnp.concatenate([b3[0:192], jnp.tile(b3[192:256], 3)])
    out = pl.pallas_call(
        _edge_body,
        grid=(GRID,),
        in_specs=[
            pl.BlockSpec((RPB, PK * 16), lambda i: (i, 0)),
            pl.BlockSpec((RPB, PK * D_IN), lambda i: (i, 0)),
            pl.BlockSpec((RPB, PK * 4), lambda i: (i, 0)),
            full((PK * 16, PK * 64)), full((1, PK * 64)),
            full((PK * 64, PK * 64)), full((1, PK * 64)),
            full((PK * 64, PK * 384)), full((1, PK * 384)),
            full((PK * 4, PK * 32)), full((PK * 4, PK * 56)),
            full((2 * PK * 32, PK * 384)),
            full((PK * 384, PK * 56)), full((PK * 56, PK * D_MSG)),
            full((PK * 56, PK * D_MSG)), full((1, PK * D_MSG)),
        ],
        out_specs=pl.BlockSpec((RPB, PK * D_MSG), lambda i: (i, 0)),
        out_shape=jax.ShapeDtypeStruct((E // PK, PK * D_MSG), jnp.float32),
    )(ef.reshape(E // PK, PK * 16), x.reshape(E // PK, PK * D_IN),
      sh.reshape(E // PK, PK * 4),
      w1b, jnp.tile(b1, PK).reshape(1, -1),
      w2b, jnp.tile(b2, PK).reshape(1, -1),
      w3b, jnp.tile(b3p, PK).reshape(1, -1),
      jnp.asarray(_CSHM), jnp.asarray(_CY), jnp.asarray(_CMX),
      jnp.asarray(_CQ56), jnp.asarray(_COS),
      jnp.asarray(_CFV), jnp.asarray(_ONESB))
    return out.reshape(E, D_MSG)


def _finish_body(agg_ref, hp_ref, wsi, w0, b0, w1g, gm, pout, out_ref):
    agg = agg_ref[0] + agg_ref[1]
    cnt = jnp.maximum(agg[:, 32:33], 1.0)
    mean = agg[:, 0:32] / cnt
    hp = hp_ref[...]
    si = jax.lax.dot(hp, wsi[...])
    o = mean + si
    os_ = o[:, 0:8]
    ov = o[:, 8:32]
    mu = jnp.mean(os_, axis=0, keepdims=True)
    osc = os_ - mu
    var = jnp.mean(osc * osc, axis=0, keepdims=True)
    os_n = osc * lax.rsqrt(var + EPS) * w0[...] + b0[...]
    colsq = jnp.mean(ov * ov, axis=0, keepdims=True)
    n2 = jax.lax.dot(colsq, gm[...])
    ov_n = ov * lax.rsqrt(n2 + EPS) * w1g[...]
    zv = ov_n + hp[:, 8:32]
    zs = os_n + hp[:, 0:8]
    out_ref[...] = jnp.concatenate(
        [zs, jax.lax.dot(zv, pout[...])], axis=1)


@jax.jit
def _tc_finish(agg2, h_p, Wsi, bn_w0, bn_b0, w1g):
    return pl.pallas_call(
        _finish_body,
        out_shape=jax.ShapeDtypeStruct((N, D_IN), jnp.float32),
    )(agg2, h_p, Wsi, bn_w0.reshape(1, 8), bn_b0.reshape(1, 8),
      w1g.reshape(1, 24), jnp.asarray(_GM), jnp.asarray(_POUT))


def kernel(h, edge_index, edge_sh, edge_features,
           W1, b1, W2, b2, W3, b3, Ws, Wv, bn_w0, bn_b0, bn_w1):
    h = h.astype(jnp.float32)
    src = edge_index[0].astype(jnp.int32).reshape(STRIPS, STRIP)
    dst = edge_index[1].astype(jnp.int32).reshape(STRIPS, STRIP)
    h_p = h[:, jnp.asarray(_PERM)]

    x = _sc_gather(h_p, src)
    msg = _tc_edges(edge_features, x, edge_sh, W1, b1, W2, b2, W3, b3)
    zeros_nd = jnp.zeros((N, D_MSG), jnp.float32)
    agg2 = _sc_scatter(msg, dst, zeros_nd)

    isq = 1.0 / jnp.sqrt(jnp.float32(MUL))
    Wsi = jnp.zeros((32, 32), jnp.float32)
    Wsi = Wsi.at[0:8, 0:8].set(Ws * isq)
    Wsi = Wsi.at[8:32, 8:32].set(jnp.kron(jnp.eye(3, dtype=jnp.float32), Wv) * isq)
    w1g = jnp.tile(bn_w1, 3)
    return _tc_finish(agg2, h_p, Wsi, bn_w0, bn_b0, w1g)


# speedup vs baseline: 1.0127x; 1.0127x over previous
import functools

import numpy as np
import jax
import jax.numpy as jnp
from jax import lax
from jax.experimental import pallas as pl
from jax.experimental.pallas import tpu as pltpu
from jax.experimental.pallas import tpu_sc as plsc

N = 10000
E = 160000
MUL = 8
D_IN = 32
D_MSG = 48
ALPHA = float(1.0 / np.sqrt(128.0))
INV_SQRT3 = float(1.0 / np.sqrt(3.0))
EPS = 1e-5

NCORES = 2
NSUB = 16
NW = NCORES * NSUB
STRIP = 125
STRIPS = E // STRIP
STRIPS_PER_W = STRIPS // NW
GROUP = 8
GROUPS = STRIPS_PER_W // GROUP
ROWS_PER_GROUP = STRIP * GROUP
ROWS_PER_W = STRIP * STRIPS_PER_W
NODE_ROWS_PER_S = N // NSUB

PK = 4
RPB = 800
GRID = (E // PK) // RPB

HI = jax.lax.Precision.HIGHEST


def _build_consts():
    perm = list(range(8)) + [8 + 3 * i + m for m in range(3) for i in range(8)]
    eye8 = np.eye(8, dtype=np.float32)
    cay = np.zeros((4, 96), np.float32)
    for i in range(8):
        cay[0, i] = 1.0
    for m in range(3):
        for i in range(8):
            cay[1 + m, 8 + m * 8 + i] = INV_SQRT3
        for j in range(8):
            cay[1 + m, 32 + m * 8 + j] = ALPHA
    for k in range(24):
        cay[0, 64 + k] = 1.0
    cm1coef = np.zeros((4, 32), np.float32)
    for i in range(8):
        cm1coef[0, i] = 1.0
    for m in range(3):
        for i in range(8):
            cm1coef[1 + m, 8 + m * 8 + i] = INV_SQRT3
    cy = np.zeros((4, 56), np.float32)
    for m in range(3):
        for j in range(8):
            cy[1 + m, 8 + m * 8 + j] = ALPHA
    for k in range(24):
        cy[0, 32 + k] = ALPHA
    cm1x = np.zeros((32, 384), np.float32)
    cm2x = np.zeros((32, 384), np.float32)
    for i in range(8):
        for j in range(8):
            cm1x[i, i * 8 + j] = 1.0
            cm2x[i, 128 + i * 8 + j] = 1.0
            for m in range(3):
                cm1x[8 + m * 8 + i, 64 + i * 8 + j] = 1.0
                cm2x[8 + m * 8 + i, 192 + m * 64 + i * 8 + j] = 1.0
    cq56 = np.zeros((384, 56), np.float32)
    for i in range(8):
        for j in range(8):
            cq56[0 * 64 + i * 8 + j, j] = ALPHA
            cq56[1 * 64 + i * 8 + j, j] = ALPHA
            for m in range(3):
                cq56[128 + i * 8 + j, 8 + m * 8 + j] = 1.0
                cq56[192 + m * 64 + i * 8 + j, 32 + m * 8 + j] = 1.0
    cos_ = np.zeros((56, 48), np.float32)
    cfv = np.zeros((56, 48), np.float32)
    for j in range(8):
        cos_[j, j] = 1.0
        for m in range(3):
            cfv[8 + m * 8 + j, 8 + m * 8 + j] = 1.0
            cfv[32 + m * 8 + j, 8 + m * 8 + j] = 1.0
    onesb = np.zeros((1, PK * D_MSG), np.float32)
    for k in range(PK):
        onesb[0, k * D_MSG + 32:(k + 1) * D_MSG] = 1.0
    gm = np.kron(np.full((3, 3), 1.0 / 3.0, np.float32), eye8)
    pout = np.zeros((24, 24), np.float32)
    for i in range(8):
        for m in range(3):
            pout[m * 8 + i, 3 * i + m] = 1.0

    def bd4(mat):
        return np.kron(np.eye(PK, dtype=np.float32), mat)

    cmx = np.vstack([bd4(cm1x), bd4(cm2x)])
    return (np.array(perm, np.int32), bd4(cm1coef), bd4(cy), cmx,
            bd4(cq56), bd4(cos_), bd4(cfv), onesb, gm, pout)


(_PERM, _CSHM, _CY, _CMX, _CQ56, _COS, _CFV, _ONESB,
 _GM, _POUT) = _build_consts()


def _gather_body(h_hbm, idx_hbm, x_hbm, idx_v, rows_v):
    c = lax.axis_index("c")
    s = lax.axis_index("s")
    wid = s * NCORES + c
    pltpu.sync_copy(idx_hbm.at[pl.ds(wid * STRIPS_PER_W, STRIPS_PER_W)], idx_v)

    def group(g, carry):
        for j in range(GROUP):
            pltpu.sync_copy(h_hbm.at[idx_v.at[g * GROUP + j]],
                            rows_v.at[pl.ds(j * STRIP, STRIP)])
        row0 = wid * ROWS_PER_W + g * ROWS_PER_GROUP
        pltpu.sync_copy(rows_v, x_hbm.at[pl.ds(row0, ROWS_PER_GROUP)])
        return carry

    lax.fori_loop(0, GROUPS, group, 0)


@jax.jit
def _sc_gather(h_p, src2d):
    fn = pl.kernel(
        _gather_body,
        out_type=jax.ShapeDtypeStruct((E, D_IN), jnp.float32),
        mesh=plsc.VectorSubcoreMesh(core_axis_name="c", subcore_axis_name="s",
                                    num_cores=NCORES, num_subcores=NSUB),
        scratch_types=[
            pltpu.VMEM((STRIPS_PER_W, STRIP), jnp.int32),
            pltpu.VMEM((ROWS_PER_GROUP, D_IN), jnp.float32),
        ],
        compiler_params=pltpu.CompilerParams(use_tc_tiling_on_sc=False),
    )
    return fn(h_p, src2d)


def _scatter_body(msg_hbm, dst_hbm, zeros_hbm, out_hbm, idx_v, msg_v, stage_v, acc_sh):
    c = lax.axis_index("c")
    s = lax.axis_index("s")
    wid = s * NCORES + c

    @pl.when(s == 0)
    def _():
        pltpu.sync_copy(zeros_hbm, acc_sh)

    plsc.subcore_barrier()
    pltpu.sync_copy(dst_hbm.at[pl.ds(wid * STRIPS_PER_W, STRIPS_PER_W)], idx_v)

    def group(g, carry):
        row0 = wid * ROWS_PER_W + g * ROWS_PER_GROUP
        pltpu.sync_copy(msg_hbm.at[pl.ds(row0, ROWS_PER_GROUP)], msg_v)
        for j in range(GROUP):
            pltpu.sync_copy(msg_v.at[pl.ds(j * STRIP, STRIP)],
                            acc_sh.at[idx_v.at[g * GROUP + j]], add=True)
        return carry

    lax.fori_loop(0, GROUPS, group, 0)
    plsc.subcore_barrier()
    pltpu.sync_copy(acc_sh.at[pl.ds(s * NODE_ROWS_PER_S, NODE_ROWS_PER_S)], stage_v)
    pltpu.sync_copy(stage_v, out_hbm.at[c, pl.ds(s * NODE_ROWS_PER_S, NODE_ROWS_PER_S)])


@jax.jit
def _sc_scatter(msg, dst2d, zeros_nd):
    fn = pl.kernel(
        _scatter_body,
        out_type=jax.ShapeDtypeStruct((NCORES, N, D_MSG), jnp.float32),
        mesh=plsc.VectorSubcoreMesh(core_axis_name="c", subcore_axis_name="s",
                                    num_cores=NCORES, num_subcores=NSUB),
        scratch_types=[
            pltpu.VMEM((STRIPS_PER_W, STRIP), jnp.int32),
            pltpu.VMEM((ROWS_PER_GROUP, D_MSG), jnp.float32),
            pltpu.VMEM((NODE_ROWS_PER_S, D_MSG), jnp.float32),
            pltpu.VMEM_SHARED((N, D_MSG), jnp.float32),
        ],
        compiler_params=pltpu.CompilerParams(use_tc_tiling_on_sc=False),
    )
    return fn(msg, dst2d, zeros_nd)


def _edge_body(ef_ref, x_ref, sh_ref, w1, b1, w2, b2, w3, b3,
               cshm, cy, cmx, cq56, cos_, cfv, onesb, out_ref):
    ef = ef_ref[...]
    xx = x_ref[...]
    shv = sh_ref[...]
    a = ef @ w1[...] + b1[...]
    a = a * jax.nn.sigmoid(a)
    a = a @ w2[...] + b2[...]
    a = a * jax.nn.sigmoid(a)
    w = a @ w3[...] + b3[...]

    shm = shv @ cshm[...]
    ymul = shv @ cy[...]
    m1 = xx * shm
    us = jnp.concatenate([m1, xx], axis=1) @ cmx[...]
    red = (w * us) @ cq56[...]
    out = red @ cos_[...] + (red * ymul) @ cfv[...] + onesb[...]
    out_ref[...] = out


@jax.jit
def _tc_edges(ef, x, sh, W1, b1, W2, b2, W3, b3):
    full = lambda shape: pl.BlockSpec(shape, lambda i: (0, 0))
    eye = jnp.eye(PK, dtype=jnp.float32)
    w1b = jnp.kron(eye, W1)
    w2b = jnp.kron(eye, W2)
    w3p = jnp.concatenate([W3[:, 0:192], jnp.tile(W3[:, 192:256], (1, 3))], axis=1)
    w3b = jnp.kron(eye, w3p)
    b3p = jnp.concatenate([b3[0:192], jnp.tile(b3[192:256], 3)])
    out = pl.pallas_call(
        _edge_body,
        grid=(GRID,),
        in_specs=[
            pl.BlockSpec((RPB, PK * 16), lambda i: (i, 0)),
            pl.BlockSpec((RPB, PK * D_IN), lambda i: (i, 0)),
            pl.BlockSpec((RPB, PK * 4), lambda i: (i, 0)),
            full((PK * 16, PK * 64)), full((1, PK * 64)),
            full((PK * 64, PK * 64)), full((1, PK * 64)),
            full((PK * 64, PK * 384)), full((1, PK * 384)),
            full((PK * 4, PK * 32)), full((PK * 4, PK * 56)),
            full((2 * PK * 32, PK * 384)),
            full((PK * 384, PK * 56)), full((PK * 56, PK * D_MSG)),
            full((PK * 56, PK * D_MSG)), full((1, PK * D_MSG)),
        ],
        out_specs=pl.BlockSpec((RPB, PK * D_MSG), lambda i: (i, 0)),
        out_shape=jax.ShapeDtypeStruct((E // PK, PK * D_MSG), jnp.float32),
    )(ef.reshape(E // PK, PK * 16), x.reshape(E // PK, PK * D_IN),
      sh.reshape(E // PK, PK * 4),
      w1b, jnp.tile(b1, PK).reshape(1, -1),
      w2b, jnp.tile(b2, PK).reshape(1, -1),
      w3b, jnp.tile(b3p, PK).reshape(1, -1),
      jnp.asarray(_CSHM), jnp.asarray(_CY), jnp.asarray(_CMX),
      jnp.asarray(_CQ56), jnp.asarray(_COS),
      jnp.asarray(_CFV), jnp.asarray(_ONESB))
    return out.reshape(E, D_MSG)


def _finish_body(agg_ref, hp_ref, wsi, w0, b0, w1g, gm, pout, out_ref):
    agg = agg_ref[0] + agg_ref[1]
    cnt = jnp.maximum(agg[:, 32:33], 1.0)
    mean = agg[:, 0:32] / cnt
    hp = hp_ref[...]
    si = jax.lax.dot(hp, wsi[...])
    o = mean + si
    os_ = o[:, 0:8]
    ov = o[:, 8:32]
    mu = jnp.mean(os_, axis=0, keepdims=True)
    osc = os_ - mu
    var = jnp.mean(osc * osc, axis=0, keepdims=True)
    os_n = osc * lax.rsqrt(var + EPS) * w0[...] + b0[...]
    colsq = jnp.mean(ov * ov, axis=0, keepdims=True)
    n2 = jax.lax.dot(colsq, gm[...])
    ov_n = ov * lax.rsqrt(n2 + EPS) * w1g[...]
    zv = ov_n + hp[:, 8:32]
    zs = os_n + hp[:, 0:8]
    out_ref[...] = jnp.concatenate(
        [zs, jax.lax.dot(zv, pout[...])], axis=1)


@jax.jit
def _tc_finish(agg2, h_p, Wsi, bn_w0, bn_b0, w1g):
    return pl.pallas_call(
        _finish_body,
        out_shape=jax.ShapeDtypeStruct((N, D_IN), jnp.float32),
    )(agg2, h_p, Wsi, bn_w0.reshape(1, 8), bn_b0.reshape(1, 8),
      w1g.reshape(1, 24), jnp.asarray(_GM), jnp.asarray(_POUT))


def kernel(h, edge_index, edge_sh, edge_features,
           W1, b1, W2, b2, W3, b3, Ws, Wv, bn_w0, bn_b0, bn_w1):
    h = h.astype(jnp.float32)
    src = edge_index[0].astype(jnp.int32).reshape(STRIPS, STRIP)
    dst = edge_index[1].astype(jnp.int32).reshape(STRIPS, STRIP)
    h_p = h[:, jnp.asarray(_PERM)]

    x = _sc_gather(h_p, src)
    msg = _tc_edges(edge_features, x, edge_sh, W1, b1, W2, b2, W3, b3)
    zeros_nd = jnp.zeros((N, D_MSG), jnp.float32)
    agg2 = _sc_scatter(msg, dst, zeros_nd)

    isq = 1.0 / jnp.sqrt(jnp.float32(MUL))
    Wsi = jnp.zeros((32, 32), jnp.float32)
    Wsi = Wsi.at[0:8, 0:8].set(Ws * isq)
    Wsi = Wsi.at[8:32, 8:32].set(jnp.kron(jnp.eye(3, dtype=jnp.float32), Wv) * isq)
    w1g = jnp.tile(bn_w1, 3)
    return _tc_finish(agg2, h_p, Wsi, bn_w0, bn_b0, w1g)


# speedup vs baseline: 1.0404x; 1.0273x over previous
import functools

import numpy as np
import jax
import jax.numpy as jnp
from jax import lax
from jax.experimental import pallas as pl
from jax.experimental.pallas import tpu as pltpu
from jax.experimental.pallas import tpu_sc as plsc

N = 10000
E = 160000
MUL = 8
D_IN = 32
D_MSG = 48
ALPHA = float(1.0 / np.sqrt(128.0))
INV_SQRT3 = float(1.0 / np.sqrt(3.0))
EPS = 1e-5

NCORES = 2
NSUB = 16
NW = NCORES * NSUB
STRIP = 125
STRIPS = E // STRIP
STRIPS_PER_W = STRIPS // NW
GROUP = 8
GROUPS = STRIPS_PER_W // GROUP
ROWS_PER_GROUP = STRIP * GROUP
ROWS_PER_W = STRIP * STRIPS_PER_W
NODE_ROWS_PER_S = N // NSUB

PK = 4
RPB = 1600
GRID = (E // PK) // RPB

HI = jax.lax.Precision.HIGHEST


def _build_consts():
    perm = list(range(8)) + [8 + 3 * i + m for m in range(3) for i in range(8)]
    eye8 = np.eye(8, dtype=np.float32)
    cay = np.zeros((4, 96), np.float32)
    for i in range(8):
        cay[0, i] = 1.0
    for m in range(3):
        for i in range(8):
            cay[1 + m, 8 + m * 8 + i] = INV_SQRT3
        for j in range(8):
            cay[1 + m, 32 + m * 8 + j] = ALPHA
    for k in range(24):
        cay[0, 64 + k] = 1.0
    cm1coef = np.zeros((4, 32), np.float32)
    for i in range(8):
        cm1coef[0, i] = 1.0
    for m in range(3):
        for i in range(8):
            cm1coef[1 + m, 8 + m * 8 + i] = INV_SQRT3
    cy = np.zeros((4, 56), np.float32)
    for m in range(3):
        for j in range(8):
            cy[1 + m, 8 + m * 8 + j] = ALPHA
    for k in range(24):
        cy[0, 32 + k] = ALPHA
    cm1x = np.zeros((32, 384), np.float32)
    cm2x = np.zeros((32, 384), np.float32)
    for i in range(8):
        for j in range(8):
            cm1x[i, i * 8 + j] = 1.0
            cm2x[i, 128 + i * 8 + j] = 1.0
            for m in range(3):
                cm1x[8 + m * 8 + i, 64 + i * 8 + j] = 1.0
                cm2x[8 + m * 8 + i, 192 + m * 64 + i * 8 + j] = 1.0
    cq56 = np.zeros((384, 56), np.float32)
    for i in range(8):
        for j in range(8):
            cq56[0 * 64 + i * 8 + j, j] = ALPHA
            cq56[1 * 64 + i * 8 + j, j] = ALPHA
            for m in range(3):
                cq56[128 + i * 8 + j, 8 + m * 8 + j] = 1.0
                cq56[192 + m * 64 + i * 8 + j, 32 + m * 8 + j] = 1.0
    cos_ = np.zeros((56, 48), np.float32)
    cfv = np.zeros((56, 48), np.float32)
    for j in range(8):
        cos_[j, j] = 1.0
        for m in range(3):
            cfv[8 + m * 8 + j, 8 + m * 8 + j] = 1.0
            cfv[32 + m * 8 + j, 8 + m * 8 + j] = 1.0
    onesb = np.zeros((1, PK * D_MSG), np.float32)
    for k in range(PK):
        onesb[0, k * D_MSG + 32:(k + 1) * D_MSG] = 1.0
    gm = np.kron(np.full((3, 3), 1.0 / 3.0, np.float32), eye8)
    pout = np.zeros((24, 24), np.float32)
    for i in range(8):
        for m in range(3):
            pout[m * 8 + i, 3 * i + m] = 1.0

    def bd4(mat):
        return np.kron(np.eye(PK, dtype=np.float32), mat)

    cmx = np.vstack([bd4(cm1x), bd4(cm2x)])
    return (np.array(perm, np.int32), bd4(cm1coef), bd4(cy), cmx,
            bd4(cq56), bd4(cos_), bd4(cfv), onesb, gm, pout)


(_PERM, _CSHM, _CY, _CMX, _CQ56, _COS, _CFV, _ONESB,
 _GM, _POUT) = _build_consts()


def _gather_body(h_hbm, idx_hbm, x_hbm, idx_v, rows_v):
    c = lax.axis_index("c")
    s = lax.axis_index("s")
    wid = s * NCORES + c
    pltpu.sync_copy(idx_hbm.at[pl.ds(wid * STRIPS_PER_W, STRIPS_PER_W)], idx_v)

    def group(g, carry):
        for j in range(GROUP):
            pltpu.sync_copy(h_hbm.at[idx_v.at[g * GROUP + j]],
                            rows_v.at[pl.ds(j * STRIP, STRIP)])
        row0 = wid * ROWS_PER_W + g * ROWS_PER_GROUP
        pltpu.sync_copy(rows_v, x_hbm.at[pl.ds(row0, ROWS_PER_GROUP)])
        return carry

    lax.fori_loop(0, GROUPS, group, 0)


@jax.jit
def _sc_gather(h_p, src2d):
    fn = pl.kernel(
        _gather_body,
        out_type=jax.ShapeDtypeStruct((E, D_IN), jnp.float32),
        mesh=plsc.VectorSubcoreMesh(core_axis_name="c", subcore_axis_name="s",
                                    num_cores=NCORES, num_subcores=NSUB),
        scratch_types=[
            pltpu.VMEM((STRIPS_PER_W, STRIP), jnp.int32),
            pltpu.VMEM((ROWS_PER_GROUP, D_IN), jnp.float32),
        ],
        compiler_params=pltpu.CompilerParams(use_tc_tiling_on_sc=False),
    )
    return fn(h_p, src2d)


def _scatter_body(msg_hbm, dst_hbm, zeros_hbm, out_hbm, idx_v, msg_v, stage_v, acc_sh):
    c = lax.axis_index("c")
    s = lax.axis_index("s")
    wid = s * NCORES + c

    @pl.when(s == 0)
    def _():
        pltpu.sync_copy(zeros_hbm, acc_sh)

    plsc.subcore_barrier()
    pltpu.sync_copy(dst_hbm.at[pl.ds(wid * STRIPS_PER_W, STRIPS_PER_W)], idx_v)

    def group(g, carry):
        row0 = wid * ROWS_PER_W + g * ROWS_PER_GROUP
        pltpu.sync_copy(msg_hbm.at[pl.ds(row0, ROWS_PER_GROUP)], msg_v)
        for j in range(GROUP):
            pltpu.sync_copy(msg_v.at[pl.ds(j * STRIP, STRIP)],
                            acc_sh.at[idx_v.at[g * GROUP + j]], add=True)
        return carry

    lax.fori_loop(0, GROUPS, group, 0)
    plsc.subcore_barrier()
    pltpu.sync_copy(acc_sh.at[pl.ds(s * NODE_ROWS_PER_S, NODE_ROWS_PER_S)], stage_v)
    pltpu.sync_copy(stage_v, out_hbm.at[c, pl.ds(s * NODE_ROWS_PER_S, NODE_ROWS_PER_S)])


@jax.jit
def _sc_scatter(msg, dst2d, zeros_nd):
    fn = pl.kernel(
        _scatter_body,
        out_type=jax.ShapeDtypeStruct((NCORES, N, D_MSG), jnp.float32),
        mesh=plsc.VectorSubcoreMesh(core_axis_name="c", subcore_axis_name="s",
                                    num_cores=NCORES, num_subcores=NSUB),
        scratch_types=[
            pltpu.VMEM((STRIPS_PER_W, STRIP), jnp.int32),
            pltpu.VMEM((ROWS_PER_GROUP, D_MSG), jnp.float32),
            pltpu.VMEM((NODE_ROWS_PER_S, D_MSG), jnp.float32),
            pltpu.VMEM_SHARED((N, D_MSG), jnp.float32),
        ],
        compiler_params=pltpu.CompilerParams(use_tc_tiling_on_sc=False),
    )
    return fn(msg, dst2d, zeros_nd)


def _edge_body(ef_ref, x_ref, sh_ref, w1, b1, w2, b2, w3, b3,
               cshm, cy, cmx, cq56, cos_, cfv, onesb, out_ref):
    ef = ef_ref[...]
    xx = x_ref[...]
    shv = sh_ref[...]
    a = ef @ w1[...] + b1[...]
    a = a * jax.nn.sigmoid(a)
    a = a @ w2[...] + b2[...]
    a = a * jax.nn.sigmoid(a)
    w = a @ w3[...] + b3[...]

    shm = shv @ cshm[...]
    ymul = shv @ cy[...]
    m1 = xx * shm
    us = jnp.concatenate([m1, xx], axis=1) @ cmx[...]
    red = (w * us) @ cq56[...]
    out = red @ cos_[...] + (red * ymul) @ cfv[...] + onesb[...]
    out_ref[...] = out


@jax.jit
def _tc_edges(ef, x, sh, W1, b1, W2, b2, W3, b3):
    full = lambda shape: pl.BlockSpec(shape, lambda i: (0, 0))
    eye = jnp.eye(PK, dtype=jnp.float32)
    w1b = jnp.kron(eye, W1)
    w2b = jnp.kron(eye, W2)
    w3p = jnp.concatenate([W3[:, 0:192], jnp.tile(W3[:, 192:256], (1, 3))], axis=1)
    w3b = jnp.kron(eye, w3p)
    b3p = jnp.concatenate([b3[0:192], jnp.tile(b3[192:256], 3)])
    out = pl.pallas_call(
        _edge_body,
        grid=(GRID,),
        in_specs=[
            pl.BlockSpec((RPB, PK * 16), lambda i: (i, 0)),
            pl.BlockSpec((RPB, PK * D_IN), lambda i: (i, 0)),
            pl.BlockSpec((RPB, PK * 4), lambda i: (i, 0)),
            full((PK * 16, PK * 64)), full((1, PK * 64)),
            full((PK * 64, PK * 64)), full((1, PK * 64)),
            full((PK * 64, PK * 384)), full((1, PK * 384)),
            full((PK * 4, PK * 32)), full((PK * 4, PK * 56)),
            full((2 * PK * 32, PK * 384)),
            full((PK * 384, PK * 56)), full((PK * 56, PK * D_MSG)),
            full((PK * 56, PK * D_MSG)), full((1, PK * D_MSG)),
        ],
        out_specs=pl.BlockSpec((RPB, PK * D_MSG), lambda i: (i, 0)),
        out_shape=jax.ShapeDtypeStruct((E // PK, PK * D_MSG), jnp.float32),
    )(ef.reshape(E // PK, PK * 16), x.reshape(E // PK, PK * D_IN),
      sh.reshape(E // PK, PK * 4),
      w1b, jnp.tile(b1, PK).reshape(1, -1),
      w2b, jnp.tile(b2, PK).reshape(1, -1),
      w3b, jnp.tile(b3p, PK).reshape(1, -1),
      jnp.asarray(_CSHM), jnp.asarray(_CY), jnp.asarray(_CMX),
      jnp.asarray(_CQ56), jnp.asarray(_COS),
      jnp.asarray(_CFV), jnp.asarray(_ONESB))
    return out.reshape(E, D_MSG)


def _finish_body(agg_ref, hp_ref, wsi, w0, b0, w1g, gm, pout, out_ref):
    agg = agg_ref[0] + agg_ref[1]
    cnt = jnp.maximum(agg[:, 32:33], 1.0)
    mean = agg[:, 0:32] / cnt
    hp = hp_ref[...]
    si = jax.lax.dot(hp, wsi[...])
    o = mean + si
    os_ = o[:, 0:8]
    ov = o[:, 8:32]
    mu = jnp.mean(os_, axis=0, keepdims=True)
    osc = os_ - mu
    var = jnp.mean(osc * osc, axis=0, keepdims=True)
    os_n = osc * lax.rsqrt(var + EPS) * w0[...] + b0[...]
    colsq = jnp.mean(ov * ov, axis=0, keepdims=True)
    n2 = jax.lax.dot(colsq, gm[...])
    ov_n = ov * lax.rsqrt(n2 + EPS) * w1g[...]
    zv = ov_n + hp[:, 8:32]
    zs = os_n + hp[:, 0:8]
    out_ref[...] = jnp.concatenate(
        [zs, jax.lax.dot(zv, pout[...])], axis=1)


@jax.jit
def _tc_finish(agg2, h_p, Wsi, bn_w0, bn_b0, w1g):
    return pl.pallas_call(
        _finish_body,
        out_shape=jax.ShapeDtypeStruct((N, D_IN), jnp.float32),
    )(agg2, h_p, Wsi, bn_w0.reshape(1, 8), bn_b0.reshape(1, 8),
      w1g.reshape(1, 24), jnp.asarray(_GM), jnp.asarray(_POUT))


def kernel(h, edge_index, edge_sh, edge_features,
           W1, b1, W2, b2, W3, b3, Ws, Wv, bn_w0, bn_b0, bn_w1):
    h = h.astype(jnp.float32)
    src = edge_index[0].astype(jnp.int32).reshape(STRIPS, STRIP)
    dst = edge_index[1].astype(jnp.int32).reshape(STRIPS, STRIP)
    h_p = h[:, jnp.asarray(_PERM)]

    x = _sc_gather(h_p, src)
    msg = _tc_edges(edge_features, x, edge_sh, W1, b1, W2, b2, W3, b3)
    zeros_nd = jnp.zeros((N, D_MSG), jnp.float32)
    agg2 = _sc_scatter(msg, dst, zeros_nd)

    isq = 1.0 / jnp.sqrt(jnp.float32(MUL))
    Wsi = jnp.zeros((32, 32), jnp.float32)
    Wsi = Wsi.at[0:8, 0:8].set(Ws * isq)
    Wsi = Wsi.at[8:32, 8:32].set(jnp.kron(jnp.eye(3, dtype=jnp.float32), Wv) * isq)
    w1g = jnp.tile(bn_w1, 3)
    return _tc_finish(agg2, h_p, Wsi, bn_w0, bn_b0, w1g)
